# Initial kernel scaffold; baseline (speedup 1.0000x reference)
import jax
import jax.numpy as jnp
from jax.experimental import pallas as pl


def kernel(hidden_states, Wg1, Wg2, ln1, ln2, Wq, Wk, Wv, Wo, Wgate, Wup, Wdown, position_ids):
    raise NotImplementedError("write your pallas kernel here")


# speedup vs baseline: 2.6467x; 2.6467x over previous
import functools
import numpy as np
import jax
import jax.numpy as jnp
from jax.experimental import pallas as pl
from jax.experimental.pallas import tpu as pltpu

B, S, D = 1, 2048, 768
H, HKV, DH = 12, 4, 64
E, TOPK = 8, 2
FF = 3072
NA = S * TOPK
BS = 128
NPAD = NA + E * BS
NBLK = NPAD // BS
SB = 256
EPS = 1e-6
SCALE = 1.0 / float(np.sqrt(DH))


def _rope_tables_np():
    inv = 1.0 / (10000.0 ** (np.arange(0, DH, 2, dtype=np.float64) / DH))
    t = np.arange(S, dtype=np.float64)
    freqs = np.outer(t, inv)
    emb = np.concatenate([freqs, freqs], axis=-1)
    return np.cos(emb).astype(np.float32), np.sin(emb).astype(np.float32)


def _rot_perm_np(width):
    n = width // DH
    P = np.zeros((width, width), dtype=np.float32)
    half = DH // 2
    for c in range(n):
        b = c * DH
        for i in range(half):
            P[b + half + i, b + i] = -1.0
            P[b + i, b + half + i] = 1.0
    return P

_COS_NP, _SIN_NP = _rope_tables_np()
_PQ_NP = _rot_perm_np(H * DH)
_PK_NP = _rot_perm_np(HKV * DH)


def _rms(x, eps=EPS):
    v = jnp.mean(x * x, axis=-1, keepdims=True)
    return x * jax.lax.rsqrt(v + eps)


def _router_body(h_ref, wg1_ref, wg2_ref, logits_ref, w2_ref, sel2_ref):
    x = h_ref[...]
    t = jnp.dot(x, wg1_ref[...], preferred_element_type=jnp.float32)
    logits = jnp.dot(t, wg2_ref[...], preferred_element_type=jnp.float32)
    logits_ref[...] = logits
    m = jnp.max(logits, axis=-1, keepdims=True)
    p = jnp.exp(logits - m)
    rw = p / jnp.sum(p, axis=-1, keepdims=True)
    iota = jax.lax.broadcasted_iota(jnp.int32, rw.shape, 1)
    m0 = jnp.max(rw, axis=-1, keepdims=True)
    sel0 = jnp.min(jnp.where(rw >= m0, iota, E), axis=-1, keepdims=True)
    rw2 = jnp.where(iota == sel0, -1.0, rw)
    m1 = jnp.max(rw2, axis=-1, keepdims=True)
    sel1 = jnp.min(jnp.where(rw2 >= m1, iota, E), axis=-1, keepdims=True)
    den = m0 + m1 + 1e-9
    w2_ref[...] = jnp.concatenate([m0 / den, m1 / den], axis=-1)
    sel2_ref[...] = jnp.concatenate([sel0, sel1], axis=-1)


def _router(h2d, Wg1, Wg2):
    return pl.pallas_call(
        _router_body,
        grid=(S // SB,),
        in_specs=[
            pl.BlockSpec((SB, D), lambda i: (i, 0)),
            pl.BlockSpec((D, D), lambda i: (0, 0)),
            pl.BlockSpec((D, E), lambda i: (0, 0)),
        ],
        out_specs=[
            pl.BlockSpec((SB, E), lambda i: (i, 0)),
            pl.BlockSpec((SB, TOPK), lambda i: (i, 0)),
            pl.BlockSpec((SB, TOPK), lambda i: (i, 0)),
        ],
        out_shape=[
            jax.ShapeDtypeStruct((S, E), jnp.float32),
            jax.ShapeDtypeStruct((S, TOPK), jnp.float32),
            jax.ShapeDtypeStruct((S, TOPK), jnp.int32),
        ],
    )(h2d, Wg1, Wg2)


def _kv_body(h_ref, ln1_ref, wk_ref, wv_ref, cos_ref, sin_ref, pk_ref,
             k_ref, v_ref):
    x = _rms(h_ref[...]) * ln1_ref[0]
    k = jnp.dot(x, wk_ref[0], preferred_element_type=jnp.float32)
    k = k * cos_ref[...] + jnp.dot(
        k, pk_ref[...], preferred_element_type=jnp.float32) * sin_ref[...]
    k_ref[0] = k
    v_ref[0] = jnp.dot(x, wv_ref[0], preferred_element_type=jnp.float32)


def _kv_dense(h2d, ln1, Wk, Wv, cos4, sin4, Pk):
    return pl.pallas_call(
        _kv_body,
        grid=(E, S // SB),
        in_specs=[
            pl.BlockSpec((SB, D), lambda e, s: (s, 0)),
            pl.BlockSpec((1, 1, D), lambda e, s: (e, 0, 0)),
            pl.BlockSpec((1, D, HKV * DH), lambda e, s: (e, 0, 0)),
            pl.BlockSpec((1, D, HKV * DH), lambda e, s: (e, 0, 0)),
            pl.BlockSpec((SB, HKV * DH), lambda e, s: (s, 0)),
            pl.BlockSpec((SB, HKV * DH), lambda e, s: (s, 0)),
            pl.BlockSpec((HKV * DH, HKV * DH), lambda e, s: (0, 0)),
        ],
        out_specs=[
            pl.BlockSpec((1, SB, HKV * DH), lambda e, s: (e, s, 0)),
            pl.BlockSpec((1, SB, HKV * DH), lambda e, s: (e, s, 0)),
        ],
        out_shape=[
            jax.ShapeDtypeStruct((E, S, HKV * DH), jnp.float32),
            jax.ShapeDtypeStruct((E, S, HKV * DH), jnp.float32),
        ],
    )(h2d, ln1, Wk, Wv, cos4, sin4, Pk)


def _attn_body(be_ref, hs_ref, cos_ref, sin_ref, pos_ref, ln1_ref,
               wq_ref, wo_ref, pq_ref, k_ref, v_ref, y1_ref, o_ref):
    hs = hs_ref[...]
    x = _rms(hs) * ln1_ref[0]
    q = jnp.dot(x, wq_ref[0], preferred_element_type=jnp.float32)
    q = q * cos_ref[...] + jnp.dot(
        q, pq_ref[...], preferred_element_type=jnp.float32) * sin_ref[...]
    q = q * SCALE
    pos_q = pos_ref[...]
    kiota = jax.lax.broadcasted_iota(jnp.int32, (BS, S), 1)
    mask = pos_q[:, :1] >= kiota
    for hh in range(H):
        kv = hh // (H // HKV)
        qh = q[:, hh * DH:(hh + 1) * DH]
        kh = k_ref[0][:, kv * DH:(kv + 1) * DH]
        vh = v_ref[0][:, kv * DH:(kv + 1) * DH]
        s = jax.lax.dot_general(qh, kh, (((1,), (1,)), ((), ())),
                                preferred_element_type=jnp.float32)
        s = jnp.where(mask, s, -1e30)
        m = jnp.max(s, axis=-1, keepdims=True)
        p = jnp.exp(s - m)
        p = p / jnp.sum(p, axis=-1, keepdims=True)
        o_ref[:, hh * DH:(hh + 1) * DH] = jnp.dot(
            p, vh, preferred_element_type=jnp.float32)
    y1_ref[...] = hs + jnp.dot(o_ref[...], wo_ref[0],
                               preferred_element_type=jnp.float32)


def _attn_sparse(block_expert, hs, cos_s, sin_s, pos_col, ln1, Wq, Wo, Pq,
                 Kc, Vc):
    grid_spec = pltpu.PrefetchScalarGridSpec(
        num_scalar_prefetch=1,
        grid=(NBLK,),
        in_specs=[
            pl.BlockSpec((BS, D), lambda b, be: (b, 0)),
            pl.BlockSpec((BS, D), lambda b, be: (b, 0)),
            pl.BlockSpec((BS, D), lambda b, be: (b, 0)),
            pl.BlockSpec((BS, 128), lambda b, be: (b, 0)),
            pl.BlockSpec((1, 1, D), lambda b, be: (be[b], 0, 0)),
            pl.BlockSpec((1, D, H * DH), lambda b, be: (be[b], 0, 0)),
            pl.BlockSpec((1, H * DH, D), lambda b, be: (be[b], 0, 0)),
            pl.BlockSpec((H * DH, H * DH), lambda b, be: (0, 0)),
            pl.BlockSpec((1, S, HKV * DH), lambda b, be: (be[b], 0, 0)),
            pl.BlockSpec((1, S, HKV * DH), lambda b, be: (be[b], 0, 0)),
        ],
        out_specs=pl.BlockSpec((BS, D), lambda b, be: (b, 0)),
        scratch_shapes=[pltpu.VMEM((BS, H * DH), jnp.float32)],
    )
    return pl.pallas_call(
        _attn_body,
        grid_spec=grid_spec,
        out_shape=jax.ShapeDtypeStruct((NPAD, D), jnp.float32),
    )(block_expert, hs, cos_s, sin_s, pos_col, ln1, Wq, Wo, Pq, Kc, Vc)


def _mlp_body(be_ref, y1_ref, ln2_ref, wg_ref, wu_ref, wd_ref, y2_ref):
    a = y1_ref[...]
    x2 = _rms(a) * ln2_ref[0]
    g = jnp.dot(x2, wg_ref[0], preferred_element_type=jnp.float32)
    u = jnp.dot(x2, wu_ref[0], preferred_element_type=jnp.float32)
    act = (g / (1.0 + jnp.exp(-g))) * u
    y2_ref[...] = a + jnp.dot(act, wd_ref[0],
                              preferred_element_type=jnp.float32)


def _mlp_sparse(block_expert, y1, ln2, Wgate, Wup, Wdown):
    grid_spec = pltpu.PrefetchScalarGridSpec(
        num_scalar_prefetch=1,
        grid=(NBLK,),
        in_specs=[
            pl.BlockSpec((BS, D), lambda b, be: (b, 0)),
            pl.BlockSpec((1, 1, D), lambda b, be: (be[b], 0, 0)),
            pl.BlockSpec((1, D, FF), lambda b, be: (be[b], 0, 0)),
            pl.BlockSpec((1, D, FF), lambda b, be: (be[b], 0, 0)),
            pl.BlockSpec((1, FF, D), lambda b, be: (be[b], 0, 0)),
        ],
        out_specs=pl.BlockSpec((BS, D), lambda b, be: (b, 0)),
    )
    return pl.pallas_call(
        _mlp_body,
        grid_spec=grid_spec,
        out_shape=jax.ShapeDtypeStruct((NPAD, D), jnp.float32),
    )(block_expert, y1, ln2, Wgate, Wup, Wdown)


def _combine_body(g0_ref, g1_ref, w0_ref, w1_ref, out_ref):
    out_ref[...] = (g0_ref[...] * w0_ref[:, :1]
                    + g1_ref[...] * w1_ref[:, :1])


def _combine(g0, g1, w0c, w1c):
    return pl.pallas_call(
        _combine_body,
        grid=(S // SB,),
        in_specs=[
            pl.BlockSpec((SB, D), lambda i: (i, 0)),
            pl.BlockSpec((SB, D), lambda i: (i, 0)),
            pl.BlockSpec((SB, 128), lambda i: (i, 0)),
            pl.BlockSpec((SB, 128), lambda i: (i, 0)),
        ],
        out_specs=pl.BlockSpec((SB, D), lambda i: (i, 0)),
        out_shape=jax.ShapeDtypeStruct((S, D), jnp.float32),
    )(g0, g1, w0c, w1c)


@jax.jit
def kernel(hidden_states, Wg1, Wg2, ln1, ln2, Wq, Wk, Wv, Wo,
           Wgate, Wup, Wdown, position_ids):
    h2d = hidden_states[0]
    cos_t = jnp.asarray(_COS_NP)
    sin_t = jnp.asarray(_SIN_NP)
    pos = position_ids[0]
    cos_s_tab = jnp.tile(cos_t, (1, H))[pos]
    sin_s_tab = jnp.tile(sin_t, (1, H))[pos]
    cos4 = jnp.tile(cos_t, (1, HKV))[pos]
    sin4 = jnp.tile(sin_t, (1, HKV))[pos]
    Pq = jnp.asarray(_PQ_NP)
    Pk = jnp.asarray(_PK_NP)

    router_logits, w2, sel2 = _router(h2d, Wg1, Wg2)

    flat_sel = sel2.reshape(-1)
    tok = jnp.arange(NA, dtype=jnp.int32) // TOPK
    ohi = (flat_sel[:, None] ==
           jnp.arange(E, dtype=jnp.int32)[None, :]).astype(jnp.int32)
    rank = jnp.take_along_axis(jnp.cumsum(ohi, axis=0) - ohi,
                               flat_sel[:, None], axis=1)[:, 0]
    counts = jnp.sum(ohi, axis=0)
    padded = ((counts + BS - 1) // BS) * BS
    cum_pad = jnp.cumsum(padded)
    pstart = cum_pad - padded
    dest = (pstart[flat_sel] + rank).astype(jnp.int32)
    tok_sorted = jnp.zeros((NPAD,), jnp.int32).at[dest].set(tok)
    block_expert = jnp.minimum(
        jnp.searchsorted(cum_pad,
                         jnp.arange(NBLK, dtype=jnp.int32) * BS,
                         side='right'),
        E - 1).astype(jnp.int32)

    hs = h2d[tok_sorted]
    pos_sorted = pos[tok_sorted]
    cos_s = cos_s_tab[pos_sorted]
    sin_s = sin_s_tab[pos_sorted]
    pos_col = jnp.broadcast_to(pos_sorted[:, None], (NPAD, 128))

    ln1r = ln1[:, None, :]
    ln2r = ln2[:, None, :]
    Kc, Vc = _kv_dense(h2d, ln1r, Wk, Wv, cos4, sin4, Pk)
    y1 = _attn_sparse(block_expert, hs, cos_s, sin_s, pos_col, ln1r,
                      Wq, Wo, Pq, Kc, Vc)
    y2 = _mlp_sparse(block_expert, y1, ln2r, Wgate, Wup, Wdown)

    dest2 = dest.reshape(S, TOPK)
    g0 = y2[dest2[:, 0]]
    g1 = y2[dest2[:, 1]]
    w0c = jnp.broadcast_to(w2[:, :1], (S, 128))
    w1c = jnp.broadcast_to(w2[:, 1:2], (S, 128))
    final = _combine(g0, g1, w0c, w1c)

    return final[None], router_logits[None]
